# baseline (speedup 1.0000x reference)
import functools

import jax
import jax.numpy as jnp
from jax import lax
from jax.experimental import pallas as pl
from jax.experimental.pallas import tpu as pltpu
from jax.experimental.pallas import tpu_sc as plsc

BATCH = 16384
HIDDEN = 32
NC = 2
NS = 16
NW = NC * NS
B_PER_W = BATCH // NW
CHUNK = 128
N_CHUNKS = B_PER_W // CHUNK
GROUPS = B_PER_W // 16


def _bpr_body(users_h, items_h, uemb_h, iemb_h, ubias_h, ibias_h, out_h,
              uidx, iidx, urows, irows, ubr, ibr, outv, sem):
    wid = lax.axis_index("s") * NC + lax.axis_index("c")
    base = wid * N_CHUNKS

    pltpu.sync_copy(users_h.at[pl.ds(base, N_CHUNKS)], uidx)
    pltpu.sync_copy(items_h.at[pl.ds(base, N_CHUNKS)], iidx)

    copies = []
    for c in range(N_CHUNKS):
        copies.append(pltpu.async_copy(
            uemb_h.at[uidx.at[c]], urows.at[pl.ds(c * CHUNK, CHUNK)], sem))
        copies.append(pltpu.async_copy(
            iemb_h.at[iidx.at[c]], irows.at[pl.ds(c * CHUNK, CHUNK)], sem))
        copies.append(pltpu.async_copy(
            ubias_h.at[uidx.at[c]], ubr.at[pl.ds(c * CHUNK, CHUNK)], sem))
        copies.append(pltpu.async_copy(
            ibias_h.at[iidx.at[c]], ibr.at[pl.ds(c * CHUNK, CHUNK)], sem))
    for cp in copies:
        cp.wait()

    lane = lax.iota(jnp.int32, 16)

    def group(g, carry):
        rowv = g * 16 + lane
        acc0 = ubr[pl.ds(g * 16, 16)]
        acc1 = ibr[pl.ds(g * 16, 16)]
        for d in range(0, HIDDEN, 2):
            c0 = jnp.full((16,), d, jnp.int32)
            c1 = jnp.full((16,), d + 1, jnp.int32)
            acc0 += (plsc.load_gather(urows, [rowv, c0])
                     * plsc.load_gather(irows, [rowv, c0]))
            acc1 += (plsc.load_gather(urows, [rowv, c1])
                     * plsc.load_gather(irows, [rowv, c1]))
        outv[pl.ds(g * 16, 16)] = acc0 + acc1
        return carry

    lax.fori_loop(0, GROUPS, group, 0)

    pltpu.sync_copy(outv, out_h.at[pl.ds(wid * B_PER_W, B_PER_W)])


@jax.jit
def _bpr_sc(users2d, items2d, user_emb, item_emb, user_bias, item_bias):
    mesh = plsc.VectorSubcoreMesh(core_axis_name="c", subcore_axis_name="s",
                                  num_cores=NC, num_subcores=NS)
    run = functools.partial(
        pl.kernel,
        out_type=jax.ShapeDtypeStruct((BATCH,), jnp.float32),
        mesh=mesh,
        compiler_params=pltpu.CompilerParams(needs_layout_passes=False,
                                             use_tc_tiling_on_sc=False),
        scratch_types=[
            pltpu.VMEM((N_CHUNKS, CHUNK), jnp.int32),
            pltpu.VMEM((N_CHUNKS, CHUNK), jnp.int32),
            pltpu.VMEM((B_PER_W, HIDDEN), jnp.float32),
            pltpu.VMEM((B_PER_W, HIDDEN), jnp.float32),
            pltpu.VMEM((B_PER_W,), jnp.float32),
            pltpu.VMEM((B_PER_W,), jnp.float32),
            pltpu.VMEM((B_PER_W,), jnp.float32),
            pltpu.SemaphoreType.DMA,
        ],
    )(_bpr_body)
    return run(users2d, items2d, user_emb, item_emb, user_bias, item_bias)


def kernel(users, items, user_emb, item_emb, user_bias, item_bias):
    users2d = users.astype(jnp.int32).reshape(NW * N_CHUNKS, CHUNK)
    items2d = items.astype(jnp.int32).reshape(NW * N_CHUNKS, CHUNK)
    return _bpr_sc(users2d, items2d, user_emb, item_emb,
                   user_bias.reshape(-1), item_bias.reshape(-1))


# speedup vs baseline: 4.2789x; 4.2789x over previous
import functools

import jax
import jax.numpy as jnp
from jax import lax
from jax.experimental import pallas as pl
from jax.experimental.pallas import tpu as pltpu
from jax.experimental.pallas import tpu_sc as plsc

BATCH = 16384
HIDDEN = 32
NC = 2
NS = 16
NW = NC * NS
B_PER_W = BATCH // NW
CHUNK = 128
N_CHUNKS = B_PER_W // CHUNK


def _bpr_body(rows_u_h, rows_i_h, users_h, items_h, ubias_h, ibias_h, out_h,
              u2, i2, uidx, iidx, ubv, ibv, outv, sem):
    wid = lax.axis_index("s") * NC + lax.axis_index("c")
    base = wid * B_PER_W

    pltpu.sync_copy(users_h.at[pl.ds(base, B_PER_W)], uidx)
    pltpu.sync_copy(items_h.at[pl.ds(base, B_PER_W)], iidx)

    descs = [
        pltpu.make_async_copy(rows_u_h.at[pl.ds(base, B_PER_W), :], u2, sem),
        pltpu.make_async_copy(rows_i_h.at[pl.ds(base, B_PER_W), :], i2, sem),
    ]
    for c in range(N_CHUNKS):
        descs.append(pltpu.make_async_copy(
            ubias_h.at[uidx.at[pl.ds(c * CHUNK, CHUNK)]],
            ubv.at[pl.ds(c * CHUNK, CHUNK)], sem))
        descs.append(pltpu.make_async_copy(
            ibias_h.at[iidx.at[pl.ds(c * CHUNK, CHUNK)]],
            ibv.at[pl.ds(c * CHUNK, CHUNK)], sem))
    for d in descs:
        d.start()
    for d in descs:
        d.wait()

    lane = lax.iota(jnp.int32, 16)

    def dot_round(k, carry):
        rowv = k * 16 + lane
        acc0 = ubv[pl.ds(k * 16, 16)]
        acc1 = ibv[pl.ds(k * 16, 16)]
        for d in range(0, HIDDEN, 2):
            c0 = jnp.full((16,), d, jnp.int32)
            c1 = jnp.full((16,), d + 1, jnp.int32)
            acc0 += (plsc.load_gather(u2, [rowv, c0])
                     * plsc.load_gather(i2, [rowv, c0]))
            acc1 += (plsc.load_gather(u2, [rowv, c1])
                     * plsc.load_gather(i2, [rowv, c1]))
        outv[pl.ds(k * 16, 16)] = acc0 + acc1
        return carry

    lax.fori_loop(0, B_PER_W // 16, dot_round, 0, unroll=1)

    pltpu.sync_copy(outv, out_h.at[pl.ds(base, B_PER_W)])


@jax.jit
def _bpr_sc(rows_u, rows_i, users, items, ubias1, ibias1):
    mesh = plsc.VectorSubcoreMesh(core_axis_name="c", subcore_axis_name="s",
                                  num_cores=NC, num_subcores=NS)
    run = functools.partial(
        pl.kernel,
        out_type=jax.ShapeDtypeStruct((BATCH,), jnp.float32),
        mesh=mesh,
        compiler_params=pltpu.CompilerParams(needs_layout_passes=False,
                                             use_tc_tiling_on_sc=False),
        scratch_types=[
            pltpu.VMEM((B_PER_W, HIDDEN), jnp.float32),
            pltpu.VMEM((B_PER_W, HIDDEN), jnp.float32),
            pltpu.VMEM((B_PER_W,), jnp.int32),
            pltpu.VMEM((B_PER_W,), jnp.int32),
            pltpu.VMEM((B_PER_W,), jnp.float32),
            pltpu.VMEM((B_PER_W,), jnp.float32),
            pltpu.VMEM((B_PER_W,), jnp.float32),
            pltpu.SemaphoreType.DMA,
        ],
    )(_bpr_body)
    return run(rows_u, rows_i, users, items, ubias1, ibias1)


def kernel(users, items, user_emb, item_emb, user_bias, item_bias):
    users = users.astype(jnp.int32)
    items = items.astype(jnp.int32)
    rows_u = jnp.take(user_emb, users, axis=0)
    rows_i = jnp.take(item_emb, items, axis=0)
    return _bpr_sc(rows_u, rows_i, users, items,
                   user_bias.reshape(-1), item_bias.reshape(-1))


# speedup vs baseline: 4.5582x; 1.0653x over previous
import functools

import jax
import jax.numpy as jnp
from jax import lax
from jax.experimental import pallas as pl
from jax.experimental.pallas import tpu as pltpu
from jax.experimental.pallas import tpu_sc as plsc

BATCH = 16384
HIDDEN = 32
NC = 2
NS = 16
NW = NC * NS
B_PER_W = BATCH // NW
CHUNK = 128
N_CHUNKS = B_PER_W // CHUNK


def _bpr_body(rows_u_h, rows_i_h, ub_rows_h, ib_rows_h, out_h,
              u2, i2, ubv, ibv, outv, sem):
    wid = lax.axis_index("s") * NC + lax.axis_index("c")
    base = wid * B_PER_W

    descs = [
        pltpu.make_async_copy(rows_u_h.at[pl.ds(base, B_PER_W), :], u2, sem),
        pltpu.make_async_copy(rows_i_h.at[pl.ds(base, B_PER_W), :], i2, sem),
        pltpu.make_async_copy(ub_rows_h.at[pl.ds(base, B_PER_W)], ubv, sem),
        pltpu.make_async_copy(ib_rows_h.at[pl.ds(base, B_PER_W)], ibv, sem),
    ]
    for d in descs:
        d.start()
    for d in descs:
        d.wait()

    lane = lax.iota(jnp.int32, 16)

    def dot_round(k, carry):
        rowv = k * 16 + lane
        acc0 = ubv[pl.ds(k * 16, 16)]
        acc1 = ibv[pl.ds(k * 16, 16)]
        for d in range(0, HIDDEN, 2):
            c0 = jnp.full((16,), d, jnp.int32)
            c1 = jnp.full((16,), d + 1, jnp.int32)
            acc0 += (plsc.load_gather(u2, [rowv, c0])
                     * plsc.load_gather(i2, [rowv, c0]))
            acc1 += (plsc.load_gather(u2, [rowv, c1])
                     * plsc.load_gather(i2, [rowv, c1]))
        outv[pl.ds(k * 16, 16)] = acc0 + acc1
        return carry

    lax.fori_loop(0, B_PER_W // 16, dot_round, 0, unroll=1)

    pltpu.sync_copy(outv, out_h.at[pl.ds(base, B_PER_W)])


@jax.jit
def _bpr_sc(rows_u, rows_i, ub_rows, ib_rows):
    mesh = plsc.VectorSubcoreMesh(core_axis_name="c", subcore_axis_name="s",
                                  num_cores=NC, num_subcores=NS)
    run = functools.partial(
        pl.kernel,
        out_type=jax.ShapeDtypeStruct((BATCH,), jnp.float32),
        mesh=mesh,
        compiler_params=pltpu.CompilerParams(needs_layout_passes=False,
                                             use_tc_tiling_on_sc=False),
        scratch_types=[
            pltpu.VMEM((B_PER_W, HIDDEN), jnp.float32),
            pltpu.VMEM((B_PER_W, HIDDEN), jnp.float32),
            pltpu.VMEM((B_PER_W,), jnp.float32),
            pltpu.VMEM((B_PER_W,), jnp.float32),
            pltpu.VMEM((B_PER_W,), jnp.float32),
            pltpu.SemaphoreType.DMA,
        ],
    )(_bpr_body)
    return run(rows_u, rows_i, ub_rows, ib_rows)


def kernel(users, items, user_emb, item_emb, user_bias, item_bias):
    users = users.astype(jnp.int32)
    items = items.astype(jnp.int32)
    rows_u = jnp.take(user_emb, users, axis=0, mode="clip")
    rows_i = jnp.take(item_emb, items, axis=0, mode="clip")
    ub_rows = jnp.take(user_bias, users, axis=0, mode="clip")
    ib_rows = jnp.take(item_bias, items, axis=0, mode="clip")
    ub_rows, ib_rows = jax.lax.optimization_barrier((ub_rows, ib_rows))
    return _bpr_sc(rows_u, rows_i, ub_rows.reshape(-1), ib_rows.reshape(-1))


# speedup vs baseline: 5.0961x; 1.1180x over previous
import functools

import jax
import jax.numpy as jnp
from jax import lax
from jax.experimental import pallas as pl
from jax.experimental.pallas import tpu as pltpu
from jax.experimental.pallas import tpu_sc as plsc

BATCH = 16384
HIDDEN = 32
NC = 2
NS = 16
NW = NC * NS
B_PER_W = BATCH // NW
CHUNK = 128
N_CHUNKS = B_PER_W // CHUNK

_MESH = dict(core_axis_name="c", subcore_axis_name="s",
             num_cores=NC, num_subcores=NS)
_PARAMS = pltpu.CompilerParams(needs_layout_passes=False,
                               use_tc_tiling_on_sc=False)


def _dot_body(rows_u_h, rows_i_h, out_h, u2, i2, outv, sem):
    wid = lax.axis_index("s") * NC + lax.axis_index("c")
    base = wid * B_PER_W

    descs = [
        pltpu.make_async_copy(rows_u_h.at[pl.ds(base, B_PER_W), :], u2, sem),
        pltpu.make_async_copy(rows_i_h.at[pl.ds(base, B_PER_W), :], i2, sem),
    ]
    for d in descs:
        d.start()
    for d in descs:
        d.wait()

    lane = lax.iota(jnp.int32, 16)

    def dot_round(k, carry):
        rowv = k * 16 + lane
        acc0 = jnp.zeros((16,), jnp.float32)
        acc1 = jnp.zeros((16,), jnp.float32)
        for d in range(0, HIDDEN, 2):
            c0 = jnp.full((16,), d, jnp.int32)
            c1 = jnp.full((16,), d + 1, jnp.int32)
            acc0 += (plsc.load_gather(u2, [rowv, c0])
                     * plsc.load_gather(i2, [rowv, c0]))
            acc1 += (plsc.load_gather(u2, [rowv, c1])
                     * plsc.load_gather(i2, [rowv, c1]))
        outv[pl.ds(k * 16, 16)] = acc0 + acc1
        return carry

    lax.fori_loop(0, B_PER_W // 16, dot_round, 0, unroll=1)

    pltpu.sync_copy(outv, out_h.at[pl.ds(base, B_PER_W)])


def _bias_body(pred_h, users_h, items_h, ubias_h, ibias_h, out_h,
               uidx, iidx, ubv, ibv, predv, outv, sem):
    wid = lax.axis_index("s") * NC + lax.axis_index("c")
    base = wid * B_PER_W

    pltpu.sync_copy(users_h.at[pl.ds(base, B_PER_W)], uidx)
    pltpu.sync_copy(items_h.at[pl.ds(base, B_PER_W)], iidx)

    descs = [pltpu.make_async_copy(pred_h.at[pl.ds(base, B_PER_W)], predv, sem)]
    for c in range(N_CHUNKS):
        descs.append(pltpu.make_async_copy(
            ubias_h.at[uidx.at[pl.ds(c * CHUNK, CHUNK)]],
            ubv.at[pl.ds(c * CHUNK, CHUNK)], sem))
        descs.append(pltpu.make_async_copy(
            ibias_h.at[iidx.at[pl.ds(c * CHUNK, CHUNK)]],
            ibv.at[pl.ds(c * CHUNK, CHUNK)], sem))
    for d in descs:
        d.start()
    for d in descs:
        d.wait()

    def add_round(k, carry):
        s = k * 16
        outv[pl.ds(s, 16)] = (predv[pl.ds(s, 16)]
                              + ubv[pl.ds(s, 16)] + ibv[pl.ds(s, 16)])
        return carry

    lax.fori_loop(0, B_PER_W // 16, add_round, 0, unroll=1)

    pltpu.sync_copy(outv, out_h.at[pl.ds(base, B_PER_W)])


@jax.jit
def _bpr_sc(rows_u, rows_i, users, items, ubias1, ibias1):
    dot = functools.partial(
        pl.kernel,
        out_type=jax.ShapeDtypeStruct((BATCH,), jnp.float32),
        mesh=plsc.VectorSubcoreMesh(**_MESH),
        compiler_params=_PARAMS,
        scratch_types=[
            pltpu.VMEM((B_PER_W, HIDDEN), jnp.float32),
            pltpu.VMEM((B_PER_W, HIDDEN), jnp.float32),
            pltpu.VMEM((B_PER_W,), jnp.float32),
            pltpu.SemaphoreType.DMA,
        ],
    )(_dot_body)
    pred0 = dot(rows_u, rows_i)

    bias = functools.partial(
        pl.kernel,
        out_type=jax.ShapeDtypeStruct((BATCH,), jnp.float32),
        mesh=plsc.VectorSubcoreMesh(**_MESH),
        compiler_params=_PARAMS,
        scratch_types=[
            pltpu.VMEM((B_PER_W,), jnp.int32),
            pltpu.VMEM((B_PER_W,), jnp.int32),
            pltpu.VMEM((B_PER_W,), jnp.float32),
            pltpu.VMEM((B_PER_W,), jnp.float32),
            pltpu.VMEM((B_PER_W,), jnp.float32),
            pltpu.VMEM((B_PER_W,), jnp.float32),
            pltpu.SemaphoreType.DMA,
        ],
    )(_bias_body)
    return bias(pred0, users, items, ubias1, ibias1)


def kernel(users, items, user_emb, item_emb, user_bias, item_bias):
    users = users.astype(jnp.int32)
    items = items.astype(jnp.int32)
    rows_u = jnp.take(user_emb, users, axis=0, mode="clip")
    rows_i = jnp.take(item_emb, items, axis=0, mode="clip")
    return _bpr_sc(rows_u, rows_i, users, items,
                   user_bias.reshape(-1), item_bias.reshape(-1))


# speedup vs baseline: 8.1306x; 1.5954x over previous
import functools

import jax
import jax.numpy as jnp
from jax import lax
from jax.experimental import pallas as pl
from jax.experimental.pallas import tpu as pltpu
from jax.experimental.pallas import tpu_sc as plsc

BATCH = 16384
HIDDEN = 32
NC = 2
NS = 16
NW = NC * NS
B_PER_W = BATCH // NW
CHUNK = 128
N_CHUNKS = B_PER_W // CHUNK


def _bpr_body(rows_u_h, rows_i_h, users_h, items_h, bias2_h, out_h,
              u2, i2, uidx, iidx, ubv, ibv, outv, sem):
    wid = lax.axis_index("s") * NC + lax.axis_index("c")
    base = wid * B_PER_W

    pltpu.sync_copy(users_h.at[pl.ds(base, B_PER_W)], uidx)
    pltpu.sync_copy(items_h.at[pl.ds(base, B_PER_W)], iidx)

    descs = [
        pltpu.make_async_copy(rows_u_h.at[pl.ds(base, B_PER_W), :], u2, sem),
        pltpu.make_async_copy(rows_i_h.at[pl.ds(base, B_PER_W), :], i2, sem),
    ]
    for c in range(N_CHUNKS):
        descs.append(pltpu.make_async_copy(
            bias2_h.at[0].at[uidx.at[pl.ds(c * CHUNK, CHUNK)]],
            ubv.at[pl.ds(c * CHUNK, CHUNK)], sem))
        descs.append(pltpu.make_async_copy(
            bias2_h.at[1].at[iidx.at[pl.ds(c * CHUNK, CHUNK)]],
            ibv.at[pl.ds(c * CHUNK, CHUNK)], sem))
    for d in descs:
        d.start()
    for d in descs:
        d.wait()

    lane = lax.iota(jnp.int32, 16)

    def dot_round(k, carry):
        rowv = k * 16 + lane
        acc0 = ubv[pl.ds(k * 16, 16)]
        acc1 = ibv[pl.ds(k * 16, 16)]
        for d in range(0, HIDDEN, 2):
            c0 = jnp.full((16,), d, jnp.int32)
            c1 = jnp.full((16,), d + 1, jnp.int32)
            acc0 += (plsc.load_gather(u2, [rowv, c0])
                     * plsc.load_gather(i2, [rowv, c0]))
            acc1 += (plsc.load_gather(u2, [rowv, c1])
                     * plsc.load_gather(i2, [rowv, c1]))
        outv[pl.ds(k * 16, 16)] = acc0 + acc1
        return carry

    lax.fori_loop(0, B_PER_W // 16, dot_round, 0, unroll=1)

    pltpu.sync_copy(outv, out_h.at[pl.ds(base, B_PER_W)])


@jax.jit
def _bpr_sc(rows_u, rows_i, users, items, bias2):
    mesh = plsc.VectorSubcoreMesh(core_axis_name="c", subcore_axis_name="s",
                                  num_cores=NC, num_subcores=NS)
    run = functools.partial(
        pl.kernel,
        out_type=jax.ShapeDtypeStruct((BATCH,), jnp.float32),
        mesh=mesh,
        compiler_params=pltpu.CompilerParams(needs_layout_passes=False,
                                             use_tc_tiling_on_sc=False),
        scratch_types=[
            pltpu.VMEM((B_PER_W, HIDDEN), jnp.float32),
            pltpu.VMEM((B_PER_W, HIDDEN), jnp.float32),
            pltpu.VMEM((B_PER_W,), jnp.int32),
            pltpu.VMEM((B_PER_W,), jnp.int32),
            pltpu.VMEM((B_PER_W,), jnp.float32),
            pltpu.VMEM((B_PER_W,), jnp.float32),
            pltpu.VMEM((B_PER_W,), jnp.float32),
            pltpu.SemaphoreType.DMA,
        ],
    )(_bpr_body)
    return run(rows_u, rows_i, users, items, bias2)


def kernel(users, items, user_emb, item_emb, user_bias, item_bias):
    users = users.astype(jnp.int32)
    items = items.astype(jnp.int32)
    rows_u = jnp.take(user_emb, users, axis=0, mode="clip")
    rows_i = jnp.take(item_emb, items, axis=0, mode="clip")
    bias2 = jnp.concatenate([user_bias.T, item_bias.T], axis=0)
    return _bpr_sc(rows_u, rows_i, users, items, bias2)
